# Initial kernel scaffold; baseline (speedup 1.0000x reference)
import jax
import jax.numpy as jnp
from jax.experimental import pallas as pl


def kernel(array, xnew):
    raise NotImplementedError("write your pallas kernel here")


# speedup vs baseline: 5407.1213x; 5407.1213x over previous
import functools

import jax
import jax.numpy as jnp
from jax import lax
from jax.experimental import pallas as pl
from jax.experimental.pallas import tpu as pltpu
from jax.experimental.pallas import tpu_sc as plsc

_N = 65536
_DELTA = float(jnp.float32(1.0) / jnp.float32(_N - 1))
_NC = 2
_NS = 16
_NW = _NC * _NS
_L = 16

_CHUNK = 8192


def _make_interp(n_queries):
    q_tile = n_queries // _NW
    n_chunks = q_tile // _CHUNK
    mesh = plsc.VectorSubcoreMesh(core_axis_name="c", subcore_axis_name="s")

    @functools.partial(
        pl.kernel,
        out_type=jax.ShapeDtypeStruct((n_queries,), jnp.float32),
        mesh=mesh,
        scratch_types=[
            pltpu.VMEM((_N,), jnp.float32),
            pltpu.VMEM((_CHUNK,), jnp.float32),
            pltpu.VMEM((_CHUNK,), jnp.float32),
        ],
        compiler_params=pltpu.CompilerParams(needs_layout_passes=False),
    )
    def interp(table_hbm, q_hbm, out_hbm, table_v, qbuf, obuf):
        wid = lax.axis_index("s") * _NC + lax.axis_index("c")
        base = wid * q_tile
        pltpu.sync_copy(table_hbm, table_v)

        def chunk_body(ci, _):
            off = base + ci * _CHUNK
            pltpu.sync_copy(q_hbm.at[pl.ds(off, _CHUNK)], qbuf)

            def vec_body(j, _):
                xf = qbuf[pl.ds(j * _L, _L)]
                i = (xf * jnp.float32(_N - 1)).astype(jnp.int32)
                i = jnp.clip(i, 0, _N - 1)
                xi1 = (i + 1).astype(jnp.float32) * jnp.float32(_DELTA)
                i = jnp.where(xi1 <= xf, i + 1, i)
                xi = i.astype(jnp.float32) * jnp.float32(_DELTA)
                i = jnp.where(xi > xf, i - 1, i)
                i = jnp.clip(i, 0, _N - 2)
                y0 = plsc.load_gather(table_v, [i])
                y1 = plsc.load_gather(table_v, [i + 1])
                x0 = i.astype(jnp.float32) * jnp.float32(_DELTA)
                x1 = (i + 1).astype(jnp.float32) * jnp.float32(_DELTA)
                obuf[pl.ds(j * _L, _L)] = y0 + (y1 - y0) / (x1 - x0) * (xf - x0)
                return 0

            lax.fori_loop(0, _CHUNK // _L, vec_body, 0)
            pltpu.sync_copy(obuf, out_hbm.at[pl.ds(off, _CHUNK)])
            return 0

        lax.fori_loop(0, n_chunks, chunk_body, 0)

    return interp


def kernel(array, xnew):
    xf = xnew.reshape(-1)
    out = _make_interp(xf.shape[0])(array, xf)
    return out.reshape(xnew.shape)


# speedup vs baseline: 10370.2699x; 1.9179x over previous
import functools

import jax
import jax.numpy as jnp
from jax import lax
from jax.experimental import pallas as pl
from jax.experimental.pallas import tpu as pltpu
from jax.experimental.pallas import tpu_sc as plsc

_N = 65536
_NC = 2
_NS = 16
_NW = _NC * _NS
_L = 16

_CHUNK = 8192


def _make_interp(n_queries):
    q_tile = n_queries // _NW
    n_chunks = q_tile // _CHUNK
    mesh = plsc.VectorSubcoreMesh(core_axis_name="c", subcore_axis_name="s")

    @functools.partial(
        pl.kernel,
        out_type=jax.ShapeDtypeStruct((n_queries,), jnp.float32),
        mesh=mesh,
        scratch_types=[
            pltpu.VMEM((_N,), jnp.float32),
            pltpu.VMEM((_CHUNK,), jnp.float32),
            pltpu.VMEM((_CHUNK,), jnp.float32),
            pltpu.VMEM((_CHUNK,), jnp.float32),
            pltpu.VMEM((_CHUNK,), jnp.float32),
            pltpu.SemaphoreType.DMA,
            pltpu.SemaphoreType.DMA,
            pltpu.SemaphoreType.DMA,
            pltpu.SemaphoreType.DMA,
        ],
        compiler_params=pltpu.CompilerParams(needs_layout_passes=False),
    )
    def interp(
        table_hbm, q_hbm, out_hbm, table_v, qb0, qb1, ob0, ob1, qs0, qs1, os0, os1
    ):
        qbuf = (qb0, qb1)
        obuf = (ob0, ob1)
        qsem = (qs0, qs1)
        osem = (os0, os1)
        wid = lax.axis_index("s") * _NC + lax.axis_index("c")
        base = wid * q_tile

        pltpu.async_copy(q_hbm.at[pl.ds(base, _CHUNK)], qbuf[0], qsem[0])
        pltpu.sync_copy(table_hbm, table_v)

        for ci in range(n_chunks):
            b = ci & 1
            off = base + ci * _CHUNK
            if ci + 1 < n_chunks:
                pltpu.async_copy(
                    q_hbm.at[pl.ds(off + _CHUNK, _CHUNK)], qbuf[1 - b], qsem[1 - b]
                )
            pltpu.make_async_copy(
                q_hbm.at[pl.ds(off, _CHUNK)], qbuf[b], qsem[b]
            ).wait()
            if ci >= 2:
                pltpu.make_async_copy(
                    obuf[b], out_hbm.at[pl.ds(off - 2 * _CHUNK, _CHUNK)], osem[b]
                ).wait()
            qb = qbuf[b]
            ob = obuf[b]

            @plsc.parallel_loop(0, _CHUNK // _L, 1, unroll=4)
            def body(j):
                xf = qb[pl.ds(j * _L, _L)]
                t = xf * jnp.float32(_N - 1)
                i = jnp.clip(t.astype(jnp.int32), 0, _N - 2)
                frac = t - i.astype(jnp.float32)
                y0 = plsc.load_gather(table_v, [i])
                y1 = plsc.load_gather(table_v, [i + 1])
                ob[pl.ds(j * _L, _L)] = y0 + (y1 - y0) * frac

            pltpu.async_copy(obuf[b], out_hbm.at[pl.ds(off, _CHUNK)], osem[b])

        for ci in (n_chunks - 2, n_chunks - 1):
            b = ci & 1
            pltpu.make_async_copy(
                obuf[b], out_hbm.at[pl.ds(base + ci * _CHUNK, _CHUNK)], osem[b]
            ).wait()

    return interp


def kernel(array, xnew):
    xf = xnew.reshape(-1)
    out = _make_interp(xf.shape[0])(array, xf)
    return out.reshape(xnew.shape)
